# baseline (speedup 1.0000x reference)
import math

import numpy as np
import jax
import jax.numpy as jnp
from jax import lax
from jax.experimental import pallas as pl
from jax.experimental.pallas import tpu as pltpu
from jax.experimental.pallas import tpu_sc as plsc

D_MODEL = 1024
N_HEAD = 16
DT = D_MODEL // N_HEAD
SEQ = 2048
TOPU = 40
ROW_TILE = 512
COL_CHUNK = 512

_COUNTS_STR = (
    "0101223320101013011111401101120020000010202210020011210240021020"
    "0203020320100000300100100020200131220221010522102001021031122010"
    "4211302100011241111111201010001110232101002111110010211202030220"
    "0121000301231011001003210020011312003010000340330031200310100100"
    "4120221140222123113011010010502001012032002111222102222011300020"
    "1131010142110201511120110111011130130000150121421012112012410001"
    "0201011112015001001111221111410212101100022202110100101001102120"
    "1130100121101011120110010211140020311110002001102113120220221001"
    "1211320011122100001202001112300102210110101001231110110031001001"
    "0010100022201002021110201201012101122121031010103230200111010211"
    "1011000220002010231521020101010012001231002301102100131100000130"
    "3101123001012010002031132210131221510002110130201020110010111002"
    "1101110112122020111103121011110003202011021101000120011212000111"
    "1202134001011411110102102100202102100111022211312011002103211221"
    "1201011111022111110112000022131011011020002102011021010112001311"
    "2320112200110210000013041011111312201012210020020301020000120010"
    "0301201121010010002101222214040001010100230111210101011111122010"
    "0102101221200210012210112110000102200321111420030012012221201212"
    "0110122101200123210212114100122121142010000210102011123001122001"
    "0001111020200002232000020101001211212031212112111020311000101011"
    "1200001100100121011001232620220011131100232010022000021120011002"
    "2102201201203010031001110110000111300022111111211212110100003130"
    "0020202010111101011003000112101123113100100021002131221314211100"
    "1101000110010114313103310010100025101100011012013101111114212100"
    "1100201321112020000151121000101223423022010010321212111220213101"
    "1110100102001000022105310400010111011002110201100211210200130120"
    "0110042010000301210102221031010100002112100101202000000113341102"
    "0100021011102121014211113011101014113110100212111001022230011213"
    "2021100114300102002211011230010001300043250223101020102020101000"
    "0102120100112012110110101110011201113230101122200211111011000300"
    "0000030312120010001012220010011111202110211201020111001131010011"
    "2131020111112010121203102102010100103111011211001041000331002100"
)
_COUNTS = (np.frombuffer(_COUNTS_STR.encode(), np.uint8)
           - ord("0")).astype(np.float32)
_MASKBIAS = np.where(_COUNTS > 0, 0.0, -np.inf).astype(np.float32)


def _proj_kernel(q_ref, k_ref, v_ref, wq_ref, wk_ref, wv_ref,
                 bq_ref, bk_ref, bv_ref, qo_ref, ko_ref, vo_ref):
    dn = (((1,), (1,)), ((), ()))
    qo_ref[...] = jax.lax.dot_general(
        q_ref[...], wq_ref[...], dn,
        preferred_element_type=jnp.float32) + bq_ref[0][None, :]
    ko_ref[...] = jax.lax.dot_general(
        k_ref[...], wk_ref[...], dn,
        preferred_element_type=jnp.float32) + bk_ref[0][None, :]
    vo_ref[...] = jax.lax.dot_general(
        v_ref[...], wv_ref[...], dn,
        preferred_element_type=jnp.float32) + bv_ref[0][None, :]


def _stats_kernel(qe_ref, qo_ref, kp_ref, mask_ref, cnt_ref,
                  me_ref, mo_ref):
    dn = (((1,), (1,)), ((), ()))
    Qe = qe_ref[0]
    Qo = qo_ref[0]
    me = jnp.full((1, SEQ), -jnp.inf, jnp.float32)
    mo = jnp.full((1, SEQ), -jnp.inf, jnp.float32)
    ae = jnp.zeros((1, SEQ), jnp.float32)
    ao = jnp.zeros((1, SEQ), jnp.float32)
    for c in range(SEQ // COL_CHUNK):
        Kc = kp_ref[0, pl.ds(c * COL_CHUNK, COL_CHUNK), :]
        Ke = Kc[:, :DT]
        Ko = Kc[:, DT:]
        mask_c = mask_ref[pl.ds(c * COL_CHUNK, COL_CHUNK), :]
        cnt_c = cnt_ref[pl.ds(c * COL_CHUNK, COL_CHUNK), :]
        Se = jax.lax.dot_general(Ke, Qe, dn,
                                 preferred_element_type=jnp.float32)
        me = jnp.maximum(me, jnp.max(Se + mask_c, axis=0, keepdims=True))
        ae = ae + jnp.sum(Se * cnt_c, axis=0, keepdims=True)
        So = jax.lax.dot_general(Ko, Qo, dn,
                                 preferred_element_type=jnp.float32)
        mo = jnp.maximum(mo, jnp.max(So + mask_c, axis=0, keepdims=True))
        ao = ao + jnp.sum(So * cnt_c, axis=0, keepdims=True)
    me_ref[0] = me - ae * (1.0 / SEQ)
    mo_ref[0] = mo - ao * (1.0 / SEQ)


_LANES = 16
_NGRP = SEQ // (_LANES * _LANES)
_IDXPAD = 48


def _splat_min(x):
    nx = -x
    pre = plsc.cummax(nx)
    suf = jnp.flip(plsc.cummax(jnp.flip(nx, 0)), 0)
    return -jnp.maximum(pre, suf)


def _sc_select_gather(m_hbm, q2_hbm, qr_hbm, m_v, pidx_v, half_v,
                      qr2_v, out_v, sem):
    wid = lax.axis_index("s") * 2 + lax.axis_index("c")
    iota = lax.iota(jnp.int32, _LANES)
    inf = jnp.float32(jnp.inf)
    infv = jnp.full((_LANES,), inf, jnp.float32)
    big = jnp.full((_LANES,), 4 * SEQ, jnp.int32)
    lane0 = iota == 0
    zero = jnp.zeros((_LANES,), jnp.int32)

    pltpu.sync_copy(m_hbm.at[wid], m_v)
    for t in range(_IDXPAD // _LANES):
        pidx_v[pl.ds(t * _LANES, _LANES)] = zero
        half_v[pl.ds(t * _LANES, _LANES)] = zero

    def _cmin(s, acc):
        off = pl.multiple_of(s * _LANES, 8)
        return jnp.minimum(acc, m_v[pl.ds(off, _LANES)])
    colmin = lax.fori_loop(0, SEQ // _LANES, _cmin, infv)

    def _extract(_, carry):
        colmin, posv = carry
        vminv = _splat_min(colmin)
        lane = _splat_min(jnp.where(colmin == vminv, iota,
                                    jnp.int32(_LANES)))
        bestv = big
        for g in range(_NGRP):
            cidx = lane + g * (_LANES * _LANES) + iota * _LANES
            vals = plsc.load_gather(m_v, [cidx])
            bestv = jnp.minimum(bestv, jnp.where(vals == vminv, cidx, big))
        best = _splat_min(bestv)
        gidx = best + wid * SEQ
        plsc.store_scatter(pidx_v, [posv], gidx >> 1, mask=lane0)
        plsc.store_scatter(half_v, [posv], gidx & 1, mask=lane0)
        plsc.store_scatter(m_v, [best], infv, mask=lane0)
        newcol = infv
        for g in range(_NGRP):
            cidx = lane + g * (_LANES * _LANES) + iota * _LANES
            newcol = jnp.minimum(newcol, plsc.load_gather(m_v, [cidx]))
        colmin = jnp.where(iota == lane, _splat_min(newcol), colmin)
        return colmin, posv + 1

    lax.fori_loop(0, TOPU, _extract, (colmin, zero))

    pltpu.async_copy(q2_hbm.at[pidx_v], qr2_v, sem).wait()
    for j in range(TOPU):
        t, l = divmod(j, _LANES)
        hs = half_v[pl.ds(t * _LANES, _LANES)]
        hj = _splat_min(jnp.where(iota == l, hs, jnp.int32(2)))
        rowv = jnp.full((_LANES,), j, jnp.int32)
        for k2 in range(DT // _LANES):
            col = hj * DT + k2 * _LANES + iota
            v = plsc.load_gather(qr2_v, [rowv, col])
            plsc.store_scatter(out_v, [rowv, k2 * _LANES + iota], v)
    pltpu.sync_copy(out_v.at[pl.ds(0, TOPU)], qr_hbm.at[wid])


def _attn_kernel(qre_ref, qro_ref, kp_ref, vp_ref, wc_ref, bc_ref, o_ref):
    h2 = pl.program_id(1)
    dn = (((1,), (1,)), ((), ()))
    K = kp_ref[0]
    V = vp_ref[0]
    ats = []
    for qr_r, lo in ((qre_ref, 0), (qro_ref, DT)):
        Kh = K[:, lo:lo + DT]
        Vh = V[:, lo:lo + DT]
        Qr = qr_r[0]
        scores = jax.lax.dot_general(Qr, Kh, dn,
                                     preferred_element_type=jnp.float32)
        scores = scores * (1.0 / math.sqrt(DT))
        smax = jnp.max(scores, axis=1, keepdims=True)
        p = jnp.exp(scores - smax)
        p = p / jnp.sum(p, axis=1, keepdims=True)
        ats.append(jnp.dot(p, Vh, preferred_element_type=jnp.float32))
    at_pair = jnp.concatenate(ats, axis=1)
    acc = jax.lax.dot_general(at_pair, wc_ref[...], dn,
                              preferred_element_type=jnp.float32)

    @pl.when(h2 == 0)
    def _():
        o_ref[0] = acc + bc_ref[0][None, :]

    @pl.when(h2 != 0)
    def _():
        o_ref[0] = o_ref[0] + acc


def kernel(q, k, v, Wq, bq, Wk, bk, Wv, bv, Wc, bc):
    B, L, D = q.shape
    H, dt = N_HEAD, DT
    BH = B * H
    HP = H // 2
    maskcol = jnp.asarray(_MASKBIAS).reshape(SEQ, 1)
    cntcol = jnp.asarray(_COUNTS).reshape(SEQ, 1)

    qp, kp, vp = pl.pallas_call(
        _proj_kernel,
        grid=(B * L // ROW_TILE,),
        in_specs=[
            pl.BlockSpec((ROW_TILE, D), lambda i: (i, 0)),
            pl.BlockSpec((ROW_TILE, D), lambda i: (i, 0)),
            pl.BlockSpec((ROW_TILE, D), lambda i: (i, 0)),
            pl.BlockSpec((D, D), lambda i: (0, 0)),
            pl.BlockSpec((D, D), lambda i: (0, 0)),
            pl.BlockSpec((D, D), lambda i: (0, 0)),
            pl.BlockSpec((1, D), lambda i: (0, 0)),
            pl.BlockSpec((1, D), lambda i: (0, 0)),
            pl.BlockSpec((1, D), lambda i: (0, 0)),
        ],
        out_specs=[
            pl.BlockSpec((ROW_TILE, D), lambda i: (i, 0)),
            pl.BlockSpec((ROW_TILE, D), lambda i: (i, 0)),
            pl.BlockSpec((ROW_TILE, D), lambda i: (i, 0)),
        ],
        out_shape=[jax.ShapeDtypeStruct((B * L, D), jnp.float32)] * 3,
        compiler_params=pltpu.CompilerParams(
            dimension_semantics=("parallel",)),
    )(q.reshape(B * L, D), k.reshape(B * L, D), v.reshape(B * L, D),
      Wq, Wk, Wv, bq.reshape(1, D), bk.reshape(1, D), bv.reshape(1, D))

    Qh = qp.reshape(BH, L, dt)
    kp3 = kp.reshape(B, L, D)
    vp3 = vp.reshape(B, L, D)

    Me, Mo = pl.pallas_call(
        _stats_kernel,
        grid=(B, HP),
        in_specs=[
            pl.BlockSpec((1, L, dt), lambda b, p: (b * H + 2 * p, 0, 0)),
            pl.BlockSpec((1, L, dt), lambda b, p: (b * H + 2 * p + 1, 0, 0)),
            pl.BlockSpec((1, L, 2 * dt), lambda b, p: (b, 0, p)),
            pl.BlockSpec((SEQ, 1), lambda b, p: (0, 0)),
            pl.BlockSpec((SEQ, 1), lambda b, p: (0, 0)),
        ],
        out_specs=[
            pl.BlockSpec((1, 1, SEQ), lambda b, p: (b * HP + p, 0, 0)),
            pl.BlockSpec((1, 1, SEQ), lambda b, p: (b * HP + p, 0, 0)),
        ],
        out_shape=[jax.ShapeDtypeStruct((B * HP, 1, SEQ), jnp.float32)] * 2,
        compiler_params=pltpu.CompilerParams(
            dimension_semantics=("parallel", "arbitrary")),
    )(Qh, Qh, kp3, maskcol, cntcol)

    M2d = jnp.stack([Me[:, 0, :], Mo[:, 0, :]], axis=1).reshape(BH, SEQ)

    qr = pl.kernel(
        _sc_select_gather,
        out_type=jax.ShapeDtypeStruct((BH, TOPU, dt), jnp.float32),
        mesh=plsc.VectorSubcoreMesh(core_axis_name="c", subcore_axis_name="s"),
        compiler_params=pltpu.CompilerParams(needs_layout_passes=False),
        scratch_types=[
            pltpu.VMEM((SEQ,), jnp.float32),
            pltpu.VMEM((_IDXPAD,), jnp.int32),
            pltpu.VMEM((_IDXPAD,), jnp.int32),
            pltpu.VMEM((_IDXPAD, 2 * DT), jnp.float32),
            pltpu.VMEM((_IDXPAD, DT), jnp.float32),
            pltpu.SemaphoreType.DMA,
        ],
    )(M2d, Qh.reshape(BH * L // 2, 2 * dt))

    out = pl.pallas_call(
        _attn_kernel,
        grid=(B, HP),
        in_specs=[
            pl.BlockSpec((1, TOPU, dt), lambda b, p: (b * H + 2 * p, 0, 0)),
            pl.BlockSpec((1, TOPU, dt), lambda b, p: (b * H + 2 * p + 1, 0, 0)),
            pl.BlockSpec((1, L, 2 * dt), lambda b, p: (b, 0, p)),
            pl.BlockSpec((1, L, 2 * dt), lambda b, p: (b, 0, p)),
            pl.BlockSpec((D, 2 * dt), lambda b, p: (0, p)),
            pl.BlockSpec((1, D), lambda b, p: (0, 0)),
        ],
        out_specs=pl.BlockSpec((1, TOPU, D), lambda b, p: (b, 0, 0)),
        out_shape=jax.ShapeDtypeStruct((B, TOPU, D), jnp.float32),
        compiler_params=pltpu.CompilerParams(
            dimension_semantics=("parallel", "arbitrary")),
    )(qr, qr, kp3, vp3, Wc, bc.reshape(1, D))
    return out


# speedup vs baseline: 1.0047x; 1.0047x over previous
import math

import numpy as np
import jax
import jax.numpy as jnp
from jax import lax
from jax.experimental import pallas as pl
from jax.experimental.pallas import tpu as pltpu
from jax.experimental.pallas import tpu_sc as plsc

D_MODEL = 1024
N_HEAD = 16
DT = D_MODEL // N_HEAD
SEQ = 2048
TOPU = 40
ROW_TILE = 512
COL_CHUNK = 512

_COUNTS_STR = (
    "0101223320101013011111401101120020000010202210020011210240021020"
    "0203020320100000300100100020200131220221010522102001021031122010"
    "4211302100011241111111201010001110232101002111110010211202030220"
    "0121000301231011001003210020011312003010000340330031200310100100"
    "4120221140222123113011010010502001012032002111222102222011300020"
    "1131010142110201511120110111011130130000150121421012112012410001"
    "0201011112015001001111221111410212101100022202110100101001102120"
    "1130100121101011120110010211140020311110002001102113120220221001"
    "1211320011122100001202001112300102210110101001231110110031001001"
    "0010100022201002021110201201012101122121031010103230200111010211"
    "1011000220002010231521020101010012001231002301102100131100000130"
    "3101123001012010002031132210131221510002110130201020110010111002"
    "1101110112122020111103121011110003202011021101000120011212000111"
    "1202134001011411110102102100202102100111022211312011002103211221"
    "1201011111022111110112000022131011011020002102011021010112001311"
    "2320112200110210000013041011111312201012210020020301020000120010"
    "0301201121010010002101222214040001010100230111210101011111122010"
    "0102101221200210012210112110000102200321111420030012012221201212"
    "0110122101200123210212114100122121142010000210102011123001122001"
    "0001111020200002232000020101001211212031212112111020311000101011"
    "1200001100100121011001232620220011131100232010022000021120011002"
    "2102201201203010031001110110000111300022111111211212110100003130"
    "0020202010111101011003000112101123113100100021002131221314211100"
    "1101000110010114313103310010100025101100011012013101111114212100"
    "1100201321112020000151121000101223423022010010321212111220213101"
    "1110100102001000022105310400010111011002110201100211210200130120"
    "0110042010000301210102221031010100002112100101202000000113341102"
    "0100021011102121014211113011101014113110100212111001022230011213"
    "2021100114300102002211011230010001300043250223101020102020101000"
    "0102120100112012110110101110011201113230101122200211111011000300"
    "0000030312120010001012220010011111202110211201020111001131010011"
    "2131020111112010121203102102010100103111011211001041000331002100"
)
_COUNTS = (np.frombuffer(_COUNTS_STR.encode(), np.uint8)
           - ord("0")).astype(np.float32)
_MASKBIAS = np.where(_COUNTS > 0, 0.0, -np.inf).astype(np.float32)


def _proj_kernel(q_ref, k_ref, v_ref, wq_ref, wk_ref, wv_ref,
                 bq_ref, bk_ref, bv_ref, qo_ref, ko_ref, vo_ref):
    dn = (((1,), (1,)), ((), ()))
    qo_ref[...] = jax.lax.dot_general(
        q_ref[...], wq_ref[...], dn,
        preferred_element_type=jnp.float32) + bq_ref[0][None, :]
    ko_ref[...] = jax.lax.dot_general(
        k_ref[...], wk_ref[...], dn,
        preferred_element_type=jnp.float32) + bk_ref[0][None, :]
    vo_ref[...] = jax.lax.dot_general(
        v_ref[...], wv_ref[...], dn,
        preferred_element_type=jnp.float32) + bv_ref[0][None, :]


def _stats_kernel(qe_ref, qo_ref, kp_ref, mask_ref, cnt_ref,
                  me_ref, mo_ref):
    dn = (((1,), (1,)), ((), ()))
    Qe = qe_ref[0]
    Qo = qo_ref[0]
    me = jnp.full((1, SEQ), -jnp.inf, jnp.float32)
    mo = jnp.full((1, SEQ), -jnp.inf, jnp.float32)
    ae = jnp.zeros((1, SEQ), jnp.float32)
    ao = jnp.zeros((1, SEQ), jnp.float32)
    for c in range(SEQ // COL_CHUNK):
        Kc = kp_ref[0, pl.ds(c * COL_CHUNK, COL_CHUNK), :]
        Ke = Kc[:, :DT]
        Ko = Kc[:, DT:]
        mask_c = mask_ref[pl.ds(c * COL_CHUNK, COL_CHUNK), :]
        cnt_c = cnt_ref[pl.ds(c * COL_CHUNK, COL_CHUNK), :]
        Se = jax.lax.dot_general(Ke, Qe, dn,
                                 preferred_element_type=jnp.float32)
        me = jnp.maximum(me, jnp.max(Se + mask_c, axis=0, keepdims=True))
        ae = ae + jnp.sum(Se * cnt_c, axis=0, keepdims=True)
        So = jax.lax.dot_general(Ko, Qo, dn,
                                 preferred_element_type=jnp.float32)
        mo = jnp.maximum(mo, jnp.max(So + mask_c, axis=0, keepdims=True))
        ao = ao + jnp.sum(So * cnt_c, axis=0, keepdims=True)
    me_ref[0] = me - ae * (1.0 / SEQ)
    mo_ref[0] = mo - ao * (1.0 / SEQ)


_LANES = 16
_NGRP = SEQ // (_LANES * _LANES)
_IDXPAD = 48


def _splat_min(x):
    nx = -x
    pre = plsc.cummax(nx)
    suf = jnp.flip(plsc.cummax(jnp.flip(nx, 0)), 0)
    return -jnp.maximum(pre, suf)


def _sc_select_gather(m_hbm, q2_hbm, qr_hbm, m_v, pidx_v, half_v,
                      qr2_v, out_v, sem):
    wid = lax.axis_index("s") * 2 + lax.axis_index("c")
    iota = lax.iota(jnp.int32, _LANES)
    inf = jnp.float32(jnp.inf)
    infv = jnp.full((_LANES,), inf, jnp.float32)
    big = jnp.full((_LANES,), 4 * SEQ, jnp.int32)
    lane0 = iota == 0
    zero = jnp.zeros((_LANES,), jnp.int32)

    pltpu.sync_copy(m_hbm.at[wid], m_v)
    for t in range(_IDXPAD // _LANES):
        pidx_v[pl.ds(t * _LANES, _LANES)] = zero
        half_v[pl.ds(t * _LANES, _LANES)] = zero

    def _cmin(s, acc):
        off = pl.multiple_of(s * _LANES, 8)
        return jnp.minimum(acc, m_v[pl.ds(off, _LANES)])
    colmin = lax.fori_loop(0, SEQ // _LANES, _cmin, infv)

    def _extract(_, carry):
        colmin, posv = carry
        vminv = _splat_min(colmin)
        lane = _splat_min(jnp.where(colmin == vminv, iota,
                                    jnp.int32(_LANES)))
        bestv = big
        gathered = []
        for g in range(_NGRP):
            cidx = lane + g * (_LANES * _LANES) + iota * _LANES
            vals = plsc.load_gather(m_v, [cidx])
            gathered.append((cidx, vals))
            bestv = jnp.minimum(bestv, jnp.where(vals == vminv, cidx, big))
        best = _splat_min(bestv)
        gidx = best + wid * SEQ
        plsc.store_scatter(pidx_v, [posv], gidx >> 1, mask=lane0)
        plsc.store_scatter(half_v, [posv], gidx & 1, mask=lane0)
        plsc.store_scatter(m_v, [best], infv, mask=lane0)
        newcol = infv
        for cidx, vals in gathered:
            newcol = jnp.minimum(newcol, jnp.where(cidx == best, infv, vals))
        colmin = jnp.where(iota == lane, _splat_min(newcol), colmin)
        return colmin, posv + 1

    lax.fori_loop(0, TOPU, _extract, (colmin, zero))

    pltpu.async_copy(q2_hbm.at[pidx_v], qr2_v, sem).wait()
    for j in range(TOPU):
        t, l = divmod(j, _LANES)
        hs = half_v[pl.ds(t * _LANES, _LANES)]
        hj = _splat_min(jnp.where(iota == l, hs, jnp.int32(2)))
        rowv = jnp.full((_LANES,), j, jnp.int32)
        for k2 in range(DT // _LANES):
            col = hj * DT + k2 * _LANES + iota
            v = plsc.load_gather(qr2_v, [rowv, col])
            plsc.store_scatter(out_v, [rowv, k2 * _LANES + iota], v)
    pltpu.sync_copy(out_v.at[pl.ds(0, TOPU)], qr_hbm.at[wid])


def _attn_kernel(qre_ref, qro_ref, kp_ref, vp_ref, wc_ref, bc_ref, o_ref):
    h2 = pl.program_id(1)
    dn = (((1,), (1,)), ((), ()))
    K = kp_ref[0]
    V = vp_ref[0]
    ats = []
    for qr_r, lo in ((qre_ref, 0), (qro_ref, DT)):
        Kh = K[:, lo:lo + DT]
        Vh = V[:, lo:lo + DT]
        Qr = qr_r[0]
        scores = jax.lax.dot_general(Qr, Kh, dn,
                                     preferred_element_type=jnp.float32)
        scores = scores * (1.0 / math.sqrt(DT))
        smax = jnp.max(scores, axis=1, keepdims=True)
        p = jnp.exp(scores - smax)
        p = p / jnp.sum(p, axis=1, keepdims=True)
        ats.append(jnp.dot(p, Vh, preferred_element_type=jnp.float32))
    at_pair = jnp.concatenate(ats, axis=1)
    acc = jax.lax.dot_general(at_pair, wc_ref[...], dn,
                              preferred_element_type=jnp.float32)

    @pl.when(h2 == 0)
    def _():
        o_ref[0] = acc + bc_ref[0][None, :]

    @pl.when(h2 != 0)
    def _():
        o_ref[0] = o_ref[0] + acc


def kernel(q, k, v, Wq, bq, Wk, bk, Wv, bv, Wc, bc):
    B, L, D = q.shape
    H, dt = N_HEAD, DT
    BH = B * H
    HP = H // 2
    maskcol = jnp.asarray(_MASKBIAS).reshape(SEQ, 1)
    cntcol = jnp.asarray(_COUNTS).reshape(SEQ, 1)

    qp, kp, vp = pl.pallas_call(
        _proj_kernel,
        grid=(B * L // ROW_TILE,),
        in_specs=[
            pl.BlockSpec((ROW_TILE, D), lambda i: (i, 0)),
            pl.BlockSpec((ROW_TILE, D), lambda i: (i, 0)),
            pl.BlockSpec((ROW_TILE, D), lambda i: (i, 0)),
            pl.BlockSpec((D, D), lambda i: (0, 0)),
            pl.BlockSpec((D, D), lambda i: (0, 0)),
            pl.BlockSpec((D, D), lambda i: (0, 0)),
            pl.BlockSpec((1, D), lambda i: (0, 0)),
            pl.BlockSpec((1, D), lambda i: (0, 0)),
            pl.BlockSpec((1, D), lambda i: (0, 0)),
        ],
        out_specs=[
            pl.BlockSpec((ROW_TILE, D), lambda i: (i, 0)),
            pl.BlockSpec((ROW_TILE, D), lambda i: (i, 0)),
            pl.BlockSpec((ROW_TILE, D), lambda i: (i, 0)),
        ],
        out_shape=[jax.ShapeDtypeStruct((B * L, D), jnp.float32)] * 3,
        compiler_params=pltpu.CompilerParams(
            dimension_semantics=("parallel",)),
    )(q.reshape(B * L, D), k.reshape(B * L, D), v.reshape(B * L, D),
      Wq, Wk, Wv, bq.reshape(1, D), bk.reshape(1, D), bv.reshape(1, D))

    Qh = qp.reshape(BH, L, dt)
    kp3 = kp.reshape(B, L, D)
    vp3 = vp.reshape(B, L, D)

    Me, Mo = pl.pallas_call(
        _stats_kernel,
        grid=(B, HP),
        in_specs=[
            pl.BlockSpec((1, L, dt), lambda b, p: (b * H + 2 * p, 0, 0)),
            pl.BlockSpec((1, L, dt), lambda b, p: (b * H + 2 * p + 1, 0, 0)),
            pl.BlockSpec((1, L, 2 * dt), lambda b, p: (b, 0, p)),
            pl.BlockSpec((SEQ, 1), lambda b, p: (0, 0)),
            pl.BlockSpec((SEQ, 1), lambda b, p: (0, 0)),
        ],
        out_specs=[
            pl.BlockSpec((1, 1, SEQ), lambda b, p: (b * HP + p, 0, 0)),
            pl.BlockSpec((1, 1, SEQ), lambda b, p: (b * HP + p, 0, 0)),
        ],
        out_shape=[jax.ShapeDtypeStruct((B * HP, 1, SEQ), jnp.float32)] * 2,
        compiler_params=pltpu.CompilerParams(
            dimension_semantics=("parallel", "arbitrary")),
    )(Qh, Qh, kp3, maskcol, cntcol)

    M2d = jnp.stack([Me[:, 0, :], Mo[:, 0, :]], axis=1).reshape(BH, SEQ)

    qr = pl.kernel(
        _sc_select_gather,
        out_type=jax.ShapeDtypeStruct((BH, TOPU, dt), jnp.float32),
        mesh=plsc.VectorSubcoreMesh(core_axis_name="c", subcore_axis_name="s"),
        compiler_params=pltpu.CompilerParams(needs_layout_passes=False),
        scratch_types=[
            pltpu.VMEM((SEQ,), jnp.float32),
            pltpu.VMEM((_IDXPAD,), jnp.int32),
            pltpu.VMEM((_IDXPAD,), jnp.int32),
            pltpu.VMEM((_IDXPAD, 2 * DT), jnp.float32),
            pltpu.VMEM((_IDXPAD, DT), jnp.float32),
            pltpu.SemaphoreType.DMA,
        ],
    )(M2d, Qh.reshape(BH * L // 2, 2 * dt))

    out = pl.pallas_call(
        _attn_kernel,
        grid=(B, HP),
        in_specs=[
            pl.BlockSpec((1, TOPU, dt), lambda b, p: (b * H + 2 * p, 0, 0)),
            pl.BlockSpec((1, TOPU, dt), lambda b, p: (b * H + 2 * p + 1, 0, 0)),
            pl.BlockSpec((1, L, 2 * dt), lambda b, p: (b, 0, p)),
            pl.BlockSpec((1, L, 2 * dt), lambda b, p: (b, 0, p)),
            pl.BlockSpec((D, 2 * dt), lambda b, p: (0, p)),
            pl.BlockSpec((1, D), lambda b, p: (0, 0)),
        ],
        out_specs=pl.BlockSpec((1, TOPU, D), lambda b, p: (b, 0, 0)),
        out_shape=jax.ShapeDtypeStruct((B, TOPU, D), jnp.float32),
        compiler_params=pltpu.CompilerParams(
            dimension_semantics=("parallel", "arbitrary")),
    )(qr, qr, kp3, vp3, Wc, bc.reshape(1, D))
    return out
